# Initial kernel scaffold; baseline (speedup 1.0000x reference)
import jax
import jax.numpy as jnp
from jax.experimental import pallas as pl


def kernel(n_inputs, e_inputs, edge_index, W_before, b_before, W_conv, b_conv, g1, be1, g2, be2, W_after, b_after, W_h, b_h, W_out, b_out):
    raise NotImplementedError("write your pallas kernel here")


# speedup vs baseline: 1.8972x; 1.8972x over previous
import functools

import jax
import jax.numpy as jnp
from jax import lax
from jax.experimental import pallas as pl
from jax.experimental.pallas import tpu as pltpu
from jax.experimental.pallas import tpu_sc as plsc

N = 10000
E = 320000
D = 128
C2 = 256
ED = 16
EPS = 1e-5
NC, NS = 2, 16
NW = NC * NS
CB = 80
PER_TILE = E // NW
N_CHUNKS = PER_TILE // CB
NP = 10240
ROWS_PER_TILE = NP // NS
ZB = 128
EB = 5000
N_EBLK = E // EB


def _softplus(z):
    return jnp.maximum(z, 0.0) + jnp.log1p(jnp.exp(-jnp.abs(z)))


def _linear_before(n_inputs, W, b):
    def f(x_ref, w_ref, b_ref, o_ref):
        o_ref[...] = jnp.dot(x_ref[...], w_ref[...],
                             preferred_element_type=jnp.float32) + b_ref[...]
    return pl.pallas_call(
        f, out_shape=jax.ShapeDtypeStruct((N, D), jnp.float32),
    )(n_inputs, W, b.reshape(1, D))


def _node_tables(x, Wd, Ws, b):
    def f(x_ref, wd_ref, ws_ref, b_ref, a_ref, bb_ref):
        xv = x_ref[...]
        a_ref[...] = jnp.dot(xv, wd_ref[...],
                             preferred_element_type=jnp.float32) + b_ref[...]
        bb_ref[...] = jnp.dot(xv, ws_ref[...],
                              preferred_element_type=jnp.float32)
    return pl.pallas_call(
        f, out_shape=[jax.ShapeDtypeStruct((N, C2), jnp.float32)] * 2,
    )(x, Wd, Ws, b.reshape(1, C2))


def _edge_stats(G1, G2, e_inputs, We):
    def f(g1_ref, g2_ref, e_ref, we_ref, acc_ref):
        i = pl.program_id(0)
        m = (g1_ref[...] + g2_ref[...]
             + jnp.dot(e_ref[...], we_ref[...],
                       preferred_element_type=jnp.float32))
        s0 = jnp.sum(m, axis=0, keepdims=True)
        s1 = jnp.sum(m * m, axis=0, keepdims=True)
        upd = jnp.concatenate([s0, s1, jnp.zeros((6, C2), jnp.float32)], axis=0)

        @pl.when(i == 0)
        def _():
            acc_ref[...] = jnp.zeros_like(acc_ref)

        acc_ref[...] += upd

    return pl.pallas_call(
        f,
        grid=(N_EBLK,),
        in_specs=[pl.BlockSpec((EB, C2), lambda i: (i, 0)),
                  pl.BlockSpec((EB, C2), lambda i: (i, 0)),
                  pl.BlockSpec((EB, ED), lambda i: (i, 0)),
                  pl.BlockSpec((ED, C2), lambda i: (0, 0))],
        out_specs=pl.BlockSpec((8, C2), lambda i: (0, 0)),
        out_shape=jax.ShapeDtypeStruct((8, C2), jnp.float32),
    )(G1, G2, e_inputs, We)


def _edge_messages(G1, G2, e_inputs, We, sums, g1l, be1l):
    def f(g1_ref, g2_ref, e_ref, we_ref, s_ref, ga_ref, be_ref, o_ref):
        mu = s_ref[0:1, :] * (1.0 / E)
        ex2 = s_ref[1:2, :] * (1.0 / E)
        var = ex2 - mu * mu
        scale = ga_ref[...] * lax.rsqrt(var + EPS)
        shift = be_ref[...] - mu * scale
        m = (g1_ref[...] + g2_ref[...]
             + jnp.dot(e_ref[...], we_ref[...],
                       preferred_element_type=jnp.float32))
        mn = m * scale + shift
        filt = mn[:, :D]
        core = mn[:, D:]
        sg = 1.0 / (1.0 + jnp.exp(-filt))
        o_ref[...] = _softplus(core) * sg

    return pl.pallas_call(
        f,
        grid=(N_EBLK,),
        in_specs=[pl.BlockSpec((EB, C2), lambda i: (i, 0)),
                  pl.BlockSpec((EB, C2), lambda i: (i, 0)),
                  pl.BlockSpec((EB, ED), lambda i: (i, 0)),
                  pl.BlockSpec((ED, C2), lambda i: (0, 0)),
                  pl.BlockSpec((8, C2), lambda i: (0, 0)),
                  pl.BlockSpec((1, C2), lambda i: (0, 0)),
                  pl.BlockSpec((1, C2), lambda i: (0, 0))],
        out_specs=pl.BlockSpec((EB, D), lambda i: (i, 0)),
        out_shape=jax.ShapeDtypeStruct((E, D), jnp.float32),
    )(G1, G2, e_inputs, We, sums, g1l.reshape(1, C2), be1l.reshape(1, C2))


def _node_update(parts, x, g2l, be2l):
    def f(p_ref, x_ref, g_ref, b_ref, o_ref):
        agg = p_ref[0, :N, :] + p_ref[1, :N, :]
        mu = jnp.mean(agg, axis=0, keepdims=True)
        var = jnp.mean(agg * agg, axis=0, keepdims=True) - mu * mu
        z = ((agg - mu) * (g_ref[...] * lax.rsqrt(var + EPS))
             + b_ref[...] + x_ref[...])
        o_ref[...] = _softplus(z)
    return pl.pallas_call(
        f, out_shape=jax.ShapeDtypeStruct((N, D), jnp.float32),
    )(parts, x, g2l.reshape(1, D), be2l.reshape(1, D))


def _readout(x, W_after, b_after, W_h, b_h, W_out, b_out):
    def f(x_ref, wa_ref, ba_ref, wh_ref, bh_ref, wo_ref, bo_ref, o_ref):
        m = jnp.mean(x_ref[...], axis=0, keepdims=True)
        h = jnp.dot(m, wa_ref[...],
                    preferred_element_type=jnp.float32) + ba_ref[...]
        h = _softplus(h)
        for j in range(2):
            h = jnp.dot(h, wh_ref[j],
                        preferred_element_type=jnp.float32) + bh_ref[j:j + 1, :]
            h = _softplus(h)
        o_ref[...] = jnp.dot(h, wo_ref[...],
                             preferred_element_type=jnp.float32) + bo_ref[...]
    return pl.pallas_call(
        f, out_shape=jax.ShapeDtypeStruct((1, 1), jnp.float32),
    )(x, W_after, b_after.reshape(1, D), W_h, b_h, W_out, b_out.reshape(1, 1))


def _sc_gather2(A, B, dst, src):
    mesh = plsc.VectorSubcoreMesh(core_axis_name="c", subcore_axis_name="s")

    @functools.partial(
        pl.kernel, mesh=mesh,
        out_type=[jax.ShapeDtypeStruct((E, C2), jnp.float32)] * 2,
        scratch_types=[pltpu.VMEM((CB,), jnp.int32),
                       pltpu.VMEM((CB,), jnp.int32),
                       pltpu.VMEM((CB, C2), jnp.float32),
                       pltpu.VMEM((CB, C2), jnp.float32),
                       pltpu.SemaphoreType.DMA,
                       pltpu.SemaphoreType.DMA],
    )
    def k(a_hbm, b_hbm, dst_hbm, src_hbm, o1_hbm, o2_hbm,
          di_v, si_v, ra_v, rb_v, sem_a, sem_b):
        wid = lax.axis_index("s") * NC + lax.axis_index("c")
        base = wid * PER_TILE

        @pl.loop(0, N_CHUNKS)
        def _(ci):
            off = base + ci * CB
            pltpu.sync_copy(dst_hbm.at[pl.ds(off, CB)], di_v)
            pltpu.sync_copy(src_hbm.at[pl.ds(off, CB)], si_v)
            ca = pltpu.async_copy(a_hbm.at[di_v], ra_v, sem_a)
            cb = pltpu.async_copy(b_hbm.at[si_v], rb_v, sem_b)
            ca.wait()
            cb.wait()
            pltpu.sync_copy(ra_v, o1_hbm.at[pl.ds(off, CB)])
            pltpu.sync_copy(rb_v, o2_hbm.at[pl.ds(off, CB)])

    return k(A, B, dst, src)


def _sc_scatter_add(msg, dst):
    mesh = plsc.VectorSubcoreMesh(core_axis_name="c", subcore_axis_name="s")

    @functools.partial(
        pl.kernel, mesh=mesh,
        out_type=jax.ShapeDtypeStruct((NC, NP, D), jnp.float32),
        scratch_types=[pltpu.VMEM((CB,), jnp.int32),
                       pltpu.VMEM((CB, D), jnp.float32),
                       pltpu.VMEM((ZB, D), jnp.float32),
                       pltpu.VMEM_SHARED((NP, D), jnp.float32)],
    )
    def k(msg_hbm, dst_hbm, out_hbm, di_v, mv, zv, agg_sh):
        c = lax.axis_index("c")
        s = lax.axis_index("s")
        wid = s * NC + c

        @pl.loop(0, ZB)
        def _(i):
            @pl.loop(0, D // 16)
            def _(j):
                zv[i, pl.ds(j * 16, 16)] = jnp.zeros((16,), jnp.float32)

        @pl.loop(0, ROWS_PER_TILE // ZB)
        def _(kk):
            pltpu.sync_copy(zv, agg_sh.at[pl.ds(s * ROWS_PER_TILE + kk * ZB, ZB)])

        plsc.subcore_barrier()

        base = wid * PER_TILE

        @pl.loop(0, N_CHUNKS)
        def _(ci):
            off = base + ci * CB
            pltpu.sync_copy(dst_hbm.at[pl.ds(off, CB)], di_v)
            pltpu.sync_copy(msg_hbm.at[pl.ds(off, CB)], mv)
            pltpu.sync_copy(mv, agg_sh.at[di_v], add=True)

        plsc.subcore_barrier()

        pltpu.sync_copy(agg_sh.at[pl.ds(s * ROWS_PER_TILE, ROWS_PER_TILE)],
                        out_hbm.at[c, pl.ds(s * ROWS_PER_TILE, ROWS_PER_TILE)])

    return k(msg, dst)


def kernel(n_inputs, e_inputs, edge_index, W_before, b_before, W_conv, b_conv,
           g1, be1, g2, be2, W_after, b_after, W_h, b_h, W_out, b_out):
    src = edge_index[0]
    dst = edge_index[1]
    x = _linear_before(n_inputs, W_before, b_before)
    for l in range(3):
        Wd = W_conv[l, :D]
        Ws = W_conv[l, D:2 * D]
        We = W_conv[l, 2 * D:]
        A, B = _node_tables(x, Wd, Ws, b_conv[l])
        G1, G2 = _sc_gather2(A, B, dst, src)
        sums = _edge_stats(G1, G2, e_inputs, We)
        msg = _edge_messages(G1, G2, e_inputs, We, sums, g1[l], be1[l])
        parts = _sc_scatter_add(msg, dst)
        x = _node_update(parts, x, g2[l], be2[l])
    return _readout(x, W_after, b_after, W_h, b_h, W_out, b_out)


# speedup vs baseline: 3.0065x; 1.5847x over previous
import functools

import jax
import jax.numpy as jnp
from jax import lax
from jax.experimental import pallas as pl
from jax.experimental.pallas import tpu as pltpu
from jax.experimental.pallas import tpu_sc as plsc

N = 10000
E = 320000
D = 128
C2 = 256
ED = 16
EPS = 1e-5
NC, NS = 2, 16
NW = NC * NS
CB = 80
PER_TILE = E // NW
N_CHUNKS = PER_TILE // CB
NP = 10240
ROWS_PER_TILE = NP // NS
ZB = 128
EB = 5000
N_EBLK = E // EB


def _softplus(z):
    return jnp.maximum(z, 0.0) + jnp.log1p(jnp.exp(-jnp.abs(z)))


def _linear_before(n_inputs, W, b):
    def f(x_ref, w_ref, b_ref, o_ref):
        o_ref[...] = jnp.dot(x_ref[...], w_ref[...],
                             preferred_element_type=jnp.float32) + b_ref[...]
    return pl.pallas_call(
        f, out_shape=jax.ShapeDtypeStruct((N, D), jnp.float32),
    )(n_inputs, W, b.reshape(1, D))


def _pack16(v):
    fb = lax.bitcast_convert_type(v[:, :D], jnp.int32)
    cb = lax.bitcast_convert_type(v[:, D:], jnp.int32)
    return jnp.bitwise_or(jnp.bitwise_and(cb, jnp.int32(-65536)),
                          lax.shift_right_logical(fb, 16))


def _unpack16(w):
    f = lax.bitcast_convert_type(lax.shift_left(w, 16), jnp.float32)
    c = lax.bitcast_convert_type(jnp.bitwise_and(w, jnp.int32(-65536)),
                                 jnp.float32)
    return f, c


def _node_tables(x, Wd, Ws, b):
    def f(x_ref, wd_ref, ws_ref, b_ref, a_ref, bb_ref):
        xv = x_ref[...]
        a = jnp.dot(xv, wd_ref[...],
                    preferred_element_type=jnp.float32) + b_ref[...]
        bb = jnp.dot(xv, ws_ref[...], preferred_element_type=jnp.float32)
        a_ref[...] = _pack16(a)
        bb_ref[...] = _pack16(bb)
    return pl.pallas_call(
        f, out_shape=[jax.ShapeDtypeStruct((N, D), jnp.int32)] * 2,
    )(x, Wd, Ws, b.reshape(1, C2))


def _edge_m(g1_ref, g2_ref, e_ref, we_ref):
    fA, cA = _unpack16(g1_ref[...])
    fB, cB = _unpack16(g2_ref[...])
    cc = jnp.dot(e_ref[...], we_ref[...], preferred_element_type=jnp.float32)
    m_f = fA + fB + cc[:, :D]
    m_c = cA + cB + cc[:, D:]
    return m_f, m_c


def _edge_stats(G1, G2, e_inputs, We):
    def f(g1_ref, g2_ref, e_ref, we_ref, acc_ref):
        i = pl.program_id(0)
        m_f, m_c = _edge_m(g1_ref, g2_ref, e_ref, we_ref)
        upd = jnp.concatenate(
            [jnp.sum(m_f, axis=0, keepdims=True),
             jnp.sum(m_c, axis=0, keepdims=True),
             jnp.sum(m_f * m_f, axis=0, keepdims=True),
             jnp.sum(m_c * m_c, axis=0, keepdims=True),
             jnp.zeros((4, D), jnp.float32)], axis=0)

        @pl.when(i == 0)
        def _():
            acc_ref[...] = jnp.zeros_like(acc_ref)

        acc_ref[...] += upd

    return pl.pallas_call(
        f,
        grid=(N_EBLK,),
        in_specs=[pl.BlockSpec((EB, D), lambda i: (i, 0)),
                  pl.BlockSpec((EB, D), lambda i: (i, 0)),
                  pl.BlockSpec((EB, ED), lambda i: (i, 0)),
                  pl.BlockSpec((ED, C2), lambda i: (0, 0))],
        out_specs=pl.BlockSpec((8, D), lambda i: (0, 0)),
        out_shape=jax.ShapeDtypeStruct((8, D), jnp.float32),
    )(G1, G2, e_inputs, We)


def _edge_messages(G1, G2, e_inputs, We, sums, g1l, be1l):
    def f(g1_ref, g2_ref, e_ref, we_ref, s_ref, ga_ref, be_ref, o_ref):
        inv_e = 1.0 / E
        mu_f = s_ref[0:1, :] * inv_e
        mu_c = s_ref[1:2, :] * inv_e
        var_f = s_ref[2:3, :] * inv_e - mu_f * mu_f
        var_c = s_ref[3:4, :] * inv_e - mu_c * mu_c
        sc_f = ga_ref[0:1, :] * lax.rsqrt(var_f + EPS)
        sc_c = ga_ref[1:2, :] * lax.rsqrt(var_c + EPS)
        sh_f = be_ref[0:1, :] - mu_f * sc_f
        sh_c = be_ref[1:2, :] - mu_c * sc_c
        m_f, m_c = _edge_m(g1_ref, g2_ref, e_ref, we_ref)
        filt = m_f * sc_f + sh_f
        core = m_c * sc_c + sh_c
        sg = 1.0 / (1.0 + jnp.exp(-filt))
        o_ref[...] = _softplus(core) * sg

    return pl.pallas_call(
        f,
        grid=(N_EBLK,),
        in_specs=[pl.BlockSpec((EB, D), lambda i: (i, 0)),
                  pl.BlockSpec((EB, D), lambda i: (i, 0)),
                  pl.BlockSpec((EB, ED), lambda i: (i, 0)),
                  pl.BlockSpec((ED, C2), lambda i: (0, 0)),
                  pl.BlockSpec((8, D), lambda i: (0, 0)),
                  pl.BlockSpec((2, D), lambda i: (0, 0)),
                  pl.BlockSpec((2, D), lambda i: (0, 0))],
        out_specs=pl.BlockSpec((EB, D), lambda i: (i, 0)),
        out_shape=jax.ShapeDtypeStruct((E, D), jnp.float32),
    )(G1, G2, e_inputs, We, sums, g1l.reshape(2, D), be1l.reshape(2, D))


def _node_update(parts, x, g2l, be2l):
    def f(p_ref, x_ref, g_ref, b_ref, o_ref):
        agg = p_ref[0, :N, :] + p_ref[1, :N, :]
        mu = jnp.mean(agg, axis=0, keepdims=True)
        var = jnp.mean(agg * agg, axis=0, keepdims=True) - mu * mu
        z = ((agg - mu) * (g_ref[...] * lax.rsqrt(var + EPS))
             + b_ref[...] + x_ref[...])
        o_ref[...] = _softplus(z)
    return pl.pallas_call(
        f, out_shape=jax.ShapeDtypeStruct((N, D), jnp.float32),
    )(parts, x, g2l.reshape(1, D), be2l.reshape(1, D))


def _readout(x, W_after, b_after, W_h, b_h, W_out, b_out):
    def f(x_ref, wa_ref, ba_ref, wh_ref, bh_ref, wo_ref, bo_ref, o_ref):
        m = jnp.mean(x_ref[...], axis=0, keepdims=True)
        h = jnp.dot(m, wa_ref[...],
                    preferred_element_type=jnp.float32) + ba_ref[...]
        h = _softplus(h)
        for j in range(2):
            h = jnp.dot(h, wh_ref[j],
                        preferred_element_type=jnp.float32) + bh_ref[j:j + 1, :]
            h = _softplus(h)
        o_ref[...] = jnp.dot(h, wo_ref[...],
                             preferred_element_type=jnp.float32) + bo_ref[...]
    return pl.pallas_call(
        f, out_shape=jax.ShapeDtypeStruct((1, 1), jnp.float32),
    )(x, W_after, b_after.reshape(1, D), W_h, b_h, W_out, b_out.reshape(1, 1))


def _sc_gather2(A, B, dst, src):
    mesh = plsc.VectorSubcoreMesh(core_axis_name="c", subcore_axis_name="s")

    @functools.partial(
        pl.kernel, mesh=mesh,
        out_type=[jax.ShapeDtypeStruct((E, D), jnp.int32)] * 2,
        scratch_types=[pltpu.VMEM((PER_TILE,), jnp.int32),
                       pltpu.VMEM((PER_TILE,), jnp.int32),
                       pltpu.VMEM((CB, D), jnp.int32),
                       pltpu.VMEM((CB, D), jnp.int32),
                       pltpu.VMEM((CB, D), jnp.int32),
                       pltpu.VMEM((CB, D), jnp.int32),
                       pltpu.SemaphoreType.DMA,
                       pltpu.SemaphoreType.DMA,
                       pltpu.SemaphoreType.DMA,
                       pltpu.SemaphoreType.DMA],
    )
    def k(a_hbm, b_hbm, dst_hbm, src_hbm, o1_hbm, o2_hbm,
          di_v, si_v, ra0, rb0, ra1, rb1, sa0, sb0, sa1, sb1):
        wid = lax.axis_index("s") * NC + lax.axis_index("c")
        base = wid * PER_TILE
        pltpu.sync_copy(dst_hbm.at[pl.ds(base, PER_TILE)], di_v)
        pltpu.sync_copy(src_hbm.at[pl.ds(base, PER_TILE)], si_v)

        def start(ci, ra, rb, sa, sb):
            loc = ci * CB
            ca = pltpu.async_copy(a_hbm.at[di_v.at[pl.ds(loc, CB)]], ra, sa)
            cb = pltpu.async_copy(b_hbm.at[si_v.at[pl.ds(loc, CB)]], rb, sb)
            return ca, cb

        def finish(ci, ra, rb, sa, sb):
            pltpu.make_async_copy(a_hbm.at[di_v.at[pl.ds(0, CB)]], ra, sa).wait()
            pltpu.make_async_copy(b_hbm.at[si_v.at[pl.ds(0, CB)]], rb, sb).wait()
            off = base + ci * CB
            pltpu.sync_copy(ra, o1_hbm.at[pl.ds(off, CB)])
            pltpu.sync_copy(rb, o2_hbm.at[pl.ds(off, CB)])

        start(0, ra0, rb0, sa0, sb0)

        @pl.loop(0, (N_CHUNKS - 1) // 2)
        def _(kk):
            c = 2 * kk
            start(c + 1, ra1, rb1, sa1, sb1)
            finish(c, ra0, rb0, sa0, sb0)
            start(c + 2, ra0, rb0, sa0, sb0)
            finish(c + 1, ra1, rb1, sa1, sb1)

        finish(N_CHUNKS - 1, ra0, rb0, sa0, sb0)

    return k(A, B, dst, src)


def _sc_scatter_add(msg, dst):
    mesh = plsc.VectorSubcoreMesh(core_axis_name="c", subcore_axis_name="s")

    @functools.partial(
        pl.kernel, mesh=mesh,
        out_type=jax.ShapeDtypeStruct((NC, NP, D), jnp.float32),
        scratch_types=[pltpu.VMEM((CB,), jnp.int32),
                       pltpu.VMEM((CB, D), jnp.float32),
                       pltpu.VMEM((ZB, D), jnp.float32),
                       pltpu.VMEM_SHARED((NP, D), jnp.float32)],
    )
    def k(msg_hbm, dst_hbm, out_hbm, di_v, mv, zv, agg_sh):
        c = lax.axis_index("c")
        s = lax.axis_index("s")
        wid = s * NC + c

        @pl.loop(0, ZB)
        def _(i):
            @pl.loop(0, D // 16)
            def _(j):
                zv[i, pl.ds(j * 16, 16)] = jnp.zeros((16,), jnp.float32)

        @pl.loop(0, ROWS_PER_TILE // ZB)
        def _(kk):
            pltpu.sync_copy(zv, agg_sh.at[pl.ds(s * ROWS_PER_TILE + kk * ZB, ZB)])

        plsc.subcore_barrier()

        base = wid * PER_TILE

        @pl.loop(0, N_CHUNKS)
        def _(ci):
            off = base + ci * CB
            pltpu.sync_copy(dst_hbm.at[pl.ds(off, CB)], di_v)
            pltpu.sync_copy(msg_hbm.at[pl.ds(off, CB)], mv)
            pltpu.sync_copy(mv, agg_sh.at[di_v], add=True)

        plsc.subcore_barrier()

        pltpu.sync_copy(agg_sh.at[pl.ds(s * ROWS_PER_TILE, ROWS_PER_TILE)],
                        out_hbm.at[c, pl.ds(s * ROWS_PER_TILE, ROWS_PER_TILE)])

    return k(msg, dst)


def kernel(n_inputs, e_inputs, edge_index, W_before, b_before, W_conv, b_conv,
           g1, be1, g2, be2, W_after, b_after, W_h, b_h, W_out, b_out):
    src = edge_index[0]
    dst = edge_index[1]
    x = _linear_before(n_inputs, W_before, b_before)
    for l in range(3):
        Wd = W_conv[l, :D]
        Ws = W_conv[l, D:2 * D]
        We = W_conv[l, 2 * D:]
        A, B = _node_tables(x, Wd, Ws, b_conv[l])
        G1, G2 = _sc_gather2(A, B, dst, src)
        sums = _edge_stats(G1, G2, e_inputs, We)
        msg = _edge_messages(G1, G2, e_inputs, We, sums, g1[l], be1[l])
        parts = _sc_scatter_add(msg, dst)
        x = _node_update(parts, x, g2[l], be2[l])
    return _readout(x, W_after, b_after, W_h, b_h, W_out, b_out)


# speedup vs baseline: 3.5093x; 1.1672x over previous
import functools

import jax
import jax.numpy as jnp
from jax import lax
from jax.experimental import pallas as pl
from jax.experimental.pallas import tpu as pltpu
from jax.experimental.pallas import tpu_sc as plsc

N = 10000
E = 320000
D = 128
C2 = 256
ED = 16
EPS = 1e-5
NC, NS = 2, 16
NW = NC * NS
CB = 80
PER_TILE = E // NW
N_CHUNKS = PER_TILE // CB
NP = 10240
ROWS_PER_TILE = NP // NS
ZB = 128
EB = 5000
N_EBLK = E // EB


def _softplus(z):
    return jnp.maximum(z, 0.0) + jnp.log1p(jnp.exp(-jnp.abs(z)))


def _linear_before(n_inputs, W, b):
    def f(x_ref, w_ref, b_ref, o_ref):
        o_ref[...] = jnp.dot(x_ref[...], w_ref[...],
                             preferred_element_type=jnp.float32) + b_ref[...]
    return pl.pallas_call(
        f, out_shape=jax.ShapeDtypeStruct((N, D), jnp.float32),
    )(n_inputs, W, b.reshape(1, D))


def _pack16(v):
    fb = lax.bitcast_convert_type(v[:, :D], jnp.int32)
    cb = lax.bitcast_convert_type(v[:, D:], jnp.int32)
    return jnp.bitwise_or(jnp.bitwise_and(cb, jnp.int32(-65536)),
                          lax.shift_right_logical(fb, 16))


def _unpack16(w):
    f = lax.bitcast_convert_type(lax.shift_left(w, 16), jnp.float32)
    c = lax.bitcast_convert_type(jnp.bitwise_and(w, jnp.int32(-65536)),
                                 jnp.float32)
    return f, c


def _node_tables(x, Wd, Ws, b):
    def f(x_ref, wd_ref, ws_ref, b_ref, a_ref, bb_ref):
        xv = x_ref[...]
        a = jnp.dot(xv, wd_ref[...],
                    preferred_element_type=jnp.float32) + b_ref[...]
        bb = jnp.dot(xv, ws_ref[...], preferred_element_type=jnp.float32)
        a_ref[...] = _pack16(a)
        bb_ref[...] = _pack16(bb)
    return pl.pallas_call(
        f, out_shape=[jax.ShapeDtypeStruct((N, D), jnp.int32)] * 2,
    )(x, Wd, Ws, b.reshape(1, C2))


def _edge_m(g1_ref, g2_ref, e_ref, we_ref):
    fA, cA = _unpack16(g1_ref[...])
    fB, cB = _unpack16(g2_ref[...])
    cc = jnp.dot(e_ref[...], we_ref[...], preferred_element_type=jnp.float32)
    m_f = fA + fB + cc[:, :D]
    m_c = cA + cB + cc[:, D:]
    return m_f, m_c


def _edge_stats(G1, G2, e_inputs, We):
    def f(g1_ref, g2_ref, e_ref, we_ref, acc_ref):
        i = pl.program_id(0)
        m_f, m_c = _edge_m(g1_ref, g2_ref, e_ref, we_ref)
        upd = jnp.concatenate(
            [jnp.sum(m_f, axis=0, keepdims=True),
             jnp.sum(m_c, axis=0, keepdims=True),
             jnp.sum(m_f * m_f, axis=0, keepdims=True),
             jnp.sum(m_c * m_c, axis=0, keepdims=True),
             jnp.zeros((4, D), jnp.float32)], axis=0)

        @pl.when(i == 0)
        def _():
            acc_ref[...] = jnp.zeros_like(acc_ref)

        acc_ref[...] += upd

    return pl.pallas_call(
        f,
        grid=(N_EBLK,),
        in_specs=[pl.BlockSpec((EB, D), lambda i: (i, 0)),
                  pl.BlockSpec((EB, D), lambda i: (i, 0)),
                  pl.BlockSpec((EB, ED), lambda i: (i, 0)),
                  pl.BlockSpec((ED, C2), lambda i: (0, 0))],
        out_specs=pl.BlockSpec((8, D), lambda i: (0, 0)),
        out_shape=jax.ShapeDtypeStruct((8, D), jnp.float32),
    )(G1, G2, e_inputs, We)


def _edge_messages(G1, G2, e_inputs, We, sums, g1l, be1l):
    def f(g1_ref, g2_ref, e_ref, we_ref, s_ref, ga_ref, be_ref, o_ref):
        inv_e = 1.0 / E
        mu_f = s_ref[0:1, :] * inv_e
        mu_c = s_ref[1:2, :] * inv_e
        var_f = s_ref[2:3, :] * inv_e - mu_f * mu_f
        var_c = s_ref[3:4, :] * inv_e - mu_c * mu_c
        sc_f = ga_ref[0:1, :] * lax.rsqrt(var_f + EPS)
        sc_c = ga_ref[1:2, :] * lax.rsqrt(var_c + EPS)
        sh_f = be_ref[0:1, :] - mu_f * sc_f
        sh_c = be_ref[1:2, :] - mu_c * sc_c
        m_f, m_c = _edge_m(g1_ref, g2_ref, e_ref, we_ref)
        filt = m_f * sc_f + sh_f
        core = m_c * sc_c + sh_c
        sg = 1.0 / (1.0 + jnp.exp(-filt))
        o_ref[...] = _softplus(core) * sg

    return pl.pallas_call(
        f,
        grid=(N_EBLK,),
        in_specs=[pl.BlockSpec((EB, D), lambda i: (i, 0)),
                  pl.BlockSpec((EB, D), lambda i: (i, 0)),
                  pl.BlockSpec((EB, ED), lambda i: (i, 0)),
                  pl.BlockSpec((ED, C2), lambda i: (0, 0)),
                  pl.BlockSpec((8, D), lambda i: (0, 0)),
                  pl.BlockSpec((2, D), lambda i: (0, 0)),
                  pl.BlockSpec((2, D), lambda i: (0, 0))],
        out_specs=pl.BlockSpec((EB, D), lambda i: (i, 0)),
        out_shape=jax.ShapeDtypeStruct((E, D), jnp.float32),
    )(G1, G2, e_inputs, We, sums, g1l.reshape(2, D), be1l.reshape(2, D))


def _node_update(parts, x, g2l, be2l):
    def f(p_ref, x_ref, g_ref, b_ref, o_ref):
        agg = p_ref[0, :N, :] + p_ref[1, :N, :]
        mu = jnp.mean(agg, axis=0, keepdims=True)
        var = jnp.mean(agg * agg, axis=0, keepdims=True) - mu * mu
        z = ((agg - mu) * (g_ref[...] * lax.rsqrt(var + EPS))
             + b_ref[...] + x_ref[...])
        o_ref[...] = _softplus(z)
    return pl.pallas_call(
        f, out_shape=jax.ShapeDtypeStruct((N, D), jnp.float32),
    )(parts, x, g2l.reshape(1, D), be2l.reshape(1, D))


def _readout(x, W_after, b_after, W_h, b_h, W_out, b_out):
    def f(x_ref, wa_ref, ba_ref, wh_ref, bh_ref, wo_ref, bo_ref, o_ref):
        m = jnp.mean(x_ref[...], axis=0, keepdims=True)
        h = jnp.dot(m, wa_ref[...],
                    preferred_element_type=jnp.float32) + ba_ref[...]
        h = _softplus(h)
        for j in range(2):
            h = jnp.dot(h, wh_ref[j],
                        preferred_element_type=jnp.float32) + bh_ref[j:j + 1, :]
            h = _softplus(h)
        o_ref[...] = jnp.dot(h, wo_ref[...],
                             preferred_element_type=jnp.float32) + bo_ref[...]
    return pl.pallas_call(
        f, out_shape=jax.ShapeDtypeStruct((1, 1), jnp.float32),
    )(x, W_after, b_after.reshape(1, D), W_h, b_h, W_out, b_out.reshape(1, 1))


def _sc_gather2(A, B, dst, src):
    mesh = plsc.VectorSubcoreMesh(core_axis_name="c", subcore_axis_name="s")

    @functools.partial(
        pl.kernel, mesh=mesh,
        out_type=[jax.ShapeDtypeStruct((E, D), jnp.int32)] * 2,
        scratch_types=[pltpu.VMEM((PER_TILE,), jnp.int32),
                       pltpu.VMEM((PER_TILE,), jnp.int32),
                       pltpu.VMEM((CB, D), jnp.int32),
                       pltpu.VMEM((CB, D), jnp.int32),
                       pltpu.VMEM((CB, D), jnp.int32),
                       pltpu.VMEM((CB, D), jnp.int32),
                       pltpu.SemaphoreType.DMA,
                       pltpu.SemaphoreType.DMA,
                       pltpu.SemaphoreType.DMA,
                       pltpu.SemaphoreType.DMA],
    )
    def k(a_hbm, b_hbm, dst_hbm, src_hbm, o1_hbm, o2_hbm,
          di_v, si_v, ra0, rb0, ra1, rb1, sa0, sb0, sa1, sb1):
        wid = lax.axis_index("s") * NC + lax.axis_index("c")
        base = wid * PER_TILE
        pltpu.sync_copy(dst_hbm.at[pl.ds(base, PER_TILE)], di_v)
        pltpu.sync_copy(src_hbm.at[pl.ds(base, PER_TILE)], si_v)

        def start(ci, ra, rb, sa, sb):
            loc = ci * CB
            ca = pltpu.async_copy(a_hbm.at[di_v.at[pl.ds(loc, CB)]], ra, sa)
            cb = pltpu.async_copy(b_hbm.at[si_v.at[pl.ds(loc, CB)]], rb, sb)
            return ca, cb

        def finish(ci, ra, rb, sa, sb):
            pltpu.make_async_copy(a_hbm.at[di_v.at[pl.ds(0, CB)]], ra, sa).wait()
            pltpu.make_async_copy(b_hbm.at[si_v.at[pl.ds(0, CB)]], rb, sb).wait()
            off = base + ci * CB
            pltpu.sync_copy(ra, o1_hbm.at[pl.ds(off, CB)])
            pltpu.sync_copy(rb, o2_hbm.at[pl.ds(off, CB)])

        start(0, ra0, rb0, sa0, sb0)

        @pl.loop(0, (N_CHUNKS - 1) // 2)
        def _(kk):
            c = 2 * kk
            start(c + 1, ra1, rb1, sa1, sb1)
            finish(c, ra0, rb0, sa0, sb0)
            start(c + 2, ra0, rb0, sa0, sb0)
            finish(c + 1, ra1, rb1, sa1, sb1)

        finish(N_CHUNKS - 1, ra0, rb0, sa0, sb0)

    return k(A, B, dst, src)


MB = 80
N_MB = PER_TILE // MB


def _sc_scatter_add(msg, dst3):
    mesh = plsc.VectorSubcoreMesh(core_axis_name="c", subcore_axis_name="s")

    @functools.partial(
        pl.kernel, mesh=mesh,
        out_type=jax.ShapeDtypeStruct((NC, NP, D), jnp.float32),
        scratch_types=[pltpu.VMEM((N_CHUNKS, CB), jnp.int32),
                       pltpu.VMEM((MB, D), jnp.float32),
                       pltpu.VMEM((MB, D), jnp.float32),
                       pltpu.VMEM((16, D), jnp.float32),
                       pltpu.VMEM_SHARED((NP, D), jnp.float32),
                       pltpu.SemaphoreType.DMA,
                       pltpu.SemaphoreType.DMA],
    )
    def k(msg_hbm, dst_hbm, out_hbm, di_v, m0, m1, zv, agg_sh, sm0, sm1):
        c = lax.axis_index("c")
        s = lax.axis_index("s")
        wid = s * NC + c
        base = wid * PER_TILE
        pltpu.sync_copy(dst_hbm.at[wid], di_v)

        @pl.loop(0, 16)
        def _(i):
            @pl.loop(0, D // 16)
            def _(j):
                zv[i, pl.ds(j * 16, 16)] = jnp.zeros((16,), jnp.float32)

        @pl.loop(0, ROWS_PER_TILE // 16)
        def _(kk):
            pltpu.sync_copy(zv, agg_sh.at[pl.ds(s * ROWS_PER_TILE + kk * 16, 16)])

        plsc.subcore_barrier()

        def load(mb, buf, sem):
            pltpu.async_copy(msg_hbm.at[pl.ds(base + mb * MB, MB)], buf, sem)

        def drain(buf, sem):
            pltpu.make_async_copy(msg_hbm.at[pl.ds(base, MB)], buf, sem).wait()

        def scat(mb, buf):
            for j in range(MB // CB):
                pltpu.sync_copy(buf.at[pl.ds(j * CB, CB)],
                                agg_sh.at[di_v.at[mb * (MB // CB) + j]],
                                add=True)

        load(0, m0, sm0)

        @pl.loop(0, (N_MB - 1) // 2)
        def _(kk):
            mb = 2 * kk
            load(mb + 1, m1, sm1)
            drain(m0, sm0)
            scat(mb, m0)
            load(mb + 2, m0, sm0)
            drain(m1, sm1)
            scat(mb + 1, m1)

        drain(m0, sm0)
        scat(N_MB - 1, m0)

        plsc.subcore_barrier()

        pltpu.sync_copy(agg_sh.at[pl.ds(s * ROWS_PER_TILE, ROWS_PER_TILE)],
                        out_hbm.at[c, pl.ds(s * ROWS_PER_TILE, ROWS_PER_TILE)])

    return k(msg, dst3)


def kernel(n_inputs, e_inputs, edge_index, W_before, b_before, W_conv, b_conv,
           g1, be1, g2, be2, W_after, b_after, W_h, b_h, W_out, b_out):
    src = edge_index[0]
    dst = edge_index[1]
    dst3 = dst.reshape(NW, N_CHUNKS, CB)
    x = _linear_before(n_inputs, W_before, b_before)
    for l in range(3):
        Wd = W_conv[l, :D]
        Ws = W_conv[l, D:2 * D]
        We = W_conv[l, 2 * D:]
        A, B = _node_tables(x, Wd, Ws, b_conv[l])
        G1, G2 = _sc_gather2(A, B, dst, src)
        sums = _edge_stats(G1, G2, e_inputs, We)
        msg = _edge_messages(G1, G2, e_inputs, We, sums, g1[l], be1[l])
        parts = _sc_scatter_add(msg, dst3)
        x = _node_update(parts, x, g2[l], be2[l])
    return _readout(x, W_after, b_after, W_h, b_h, W_out, b_out)


# speedup vs baseline: 3.6007x; 1.0261x over previous
import functools

import jax
import jax.numpy as jnp
from jax import lax
from jax.experimental import pallas as pl
from jax.experimental.pallas import tpu as pltpu
from jax.experimental.pallas import tpu_sc as plsc

N = 10000
E = 320000
D = 128
C2 = 256
ED = 16
EPS = 1e-5
NC, NS = 2, 16
NW = NC * NS
CB = 80
PER_TILE = E // NW
N_CHUNKS = PER_TILE // CB
NP = 10240
ROWS_PER_TILE = NP // NS
ZB = 128
EB = 5000
N_EBLK = E // EB


def _softplus(z):
    return jnp.maximum(z, 0.0) + jnp.log1p(jnp.exp(-jnp.abs(z)))


def _linear_before(n_inputs, W, b):
    def f(x_ref, w_ref, b_ref, o_ref):
        o_ref[...] = jnp.dot(x_ref[...], w_ref[...],
                             preferred_element_type=jnp.float32) + b_ref[...]
    return pl.pallas_call(
        f, out_shape=jax.ShapeDtypeStruct((N, D), jnp.float32),
    )(n_inputs, W, b.reshape(1, D))


def _pack2(f, c):
    fb = lax.bitcast_convert_type(f, jnp.int32)
    cb = lax.bitcast_convert_type(c, jnp.int32)
    return jnp.bitwise_or(jnp.bitwise_and(cb, jnp.int32(-65536)),
                          lax.shift_right_logical(fb, 16))


def _pack16(v):
    return _pack2(v[:, :D], v[:, D:])


def _unpack16(w):
    f = lax.bitcast_convert_type(lax.shift_left(w, 16), jnp.float32)
    c = lax.bitcast_convert_type(jnp.bitwise_and(w, jnp.int32(-65536)),
                                 jnp.float32)
    return f, c


def _node_tables(x, Wd, Ws, b):
    def f(x_ref, wd_ref, ws_ref, b_ref, a_ref, bb_ref):
        xv = x_ref[...]
        a = jnp.dot(xv, wd_ref[...],
                    preferred_element_type=jnp.float32) + b_ref[...]
        bb = jnp.dot(xv, ws_ref[...], preferred_element_type=jnp.float32)
        a_ref[...] = _pack16(a)
        bb_ref[...] = _pack16(bb)
    return pl.pallas_call(
        f, out_shape=[jax.ShapeDtypeStruct((N, D), jnp.int32)] * 2,
    )(x, Wd, Ws, b.reshape(1, C2))


def _edge_m(g1_ref, g2_ref, e_ref, we_ref):
    fA, cA = _unpack16(g1_ref[...])
    fB, cB = _unpack16(g2_ref[...])
    cc = jnp.dot(e_ref[...], we_ref[...], preferred_element_type=jnp.float32)
    m_f = fA + fB + cc[:, :D]
    m_c = cA + cB + cc[:, D:]
    return m_f, m_c


def _edge_stats(G1, G2, e_inputs, We):
    def f(g1_ref, g2_ref, e_ref, we_ref, acc_ref, mp_ref):
        i = pl.program_id(0)
        m_f, m_c = _edge_m(g1_ref, g2_ref, e_ref, we_ref)
        mp_ref[...] = _pack2(m_f, m_c)
        upd = jnp.concatenate(
            [jnp.sum(m_f, axis=0, keepdims=True),
             jnp.sum(m_c, axis=0, keepdims=True),
             jnp.sum(m_f * m_f, axis=0, keepdims=True),
             jnp.sum(m_c * m_c, axis=0, keepdims=True),
             jnp.zeros((4, D), jnp.float32)], axis=0)

        @pl.when(i == 0)
        def _():
            acc_ref[...] = jnp.zeros_like(acc_ref)

        acc_ref[...] += upd

    return pl.pallas_call(
        f,
        grid=(N_EBLK,),
        in_specs=[pl.BlockSpec((EB, D), lambda i: (i, 0)),
                  pl.BlockSpec((EB, D), lambda i: (i, 0)),
                  pl.BlockSpec((EB, ED), lambda i: (i, 0)),
                  pl.BlockSpec((ED, C2), lambda i: (0, 0))],
        out_specs=[pl.BlockSpec((8, D), lambda i: (0, 0)),
                   pl.BlockSpec((EB, D), lambda i: (i, 0))],
        out_shape=[jax.ShapeDtypeStruct((8, D), jnp.float32),
                   jax.ShapeDtypeStruct((E, D), jnp.int32)],
    )(G1, G2, e_inputs, We)


def _edge_messages(Mp, sums, g1l, be1l):
    def f(mp_ref, s_ref, ga_ref, be_ref, o_ref):
        inv_e = 1.0 / E
        mu_f = s_ref[0:1, :] * inv_e
        mu_c = s_ref[1:2, :] * inv_e
        var_f = s_ref[2:3, :] * inv_e - mu_f * mu_f
        var_c = s_ref[3:4, :] * inv_e - mu_c * mu_c
        sc_f = ga_ref[0:1, :] * lax.rsqrt(var_f + EPS)
        sc_c = ga_ref[1:2, :] * lax.rsqrt(var_c + EPS)
        sh_f = be_ref[0:1, :] - mu_f * sc_f
        sh_c = be_ref[1:2, :] - mu_c * sc_c
        m_f, m_c = _unpack16(mp_ref[...])
        filt = m_f * sc_f + sh_f
        core = m_c * sc_c + sh_c
        sg = 1.0 / (1.0 + jnp.exp(-filt))
        o_ref[...] = _softplus(core) * sg

    return pl.pallas_call(
        f,
        grid=(N_EBLK,),
        in_specs=[pl.BlockSpec((EB, D), lambda i: (i, 0)),
                  pl.BlockSpec((8, D), lambda i: (0, 0)),
                  pl.BlockSpec((2, D), lambda i: (0, 0)),
                  pl.BlockSpec((2, D), lambda i: (0, 0))],
        out_specs=pl.BlockSpec((EB, D), lambda i: (i, 0)),
        out_shape=jax.ShapeDtypeStruct((E, D), jnp.float32),
    )(Mp, sums, g1l.reshape(2, D), be1l.reshape(2, D))


def _node_update(parts, x, g2l, be2l):
    def f(p_ref, x_ref, g_ref, b_ref, o_ref):
        agg = p_ref[0, :N, :] + p_ref[1, :N, :]
        mu = jnp.mean(agg, axis=0, keepdims=True)
        var = jnp.mean(agg * agg, axis=0, keepdims=True) - mu * mu
        z = ((agg - mu) * (g_ref[...] * lax.rsqrt(var + EPS))
             + b_ref[...] + x_ref[...])
        o_ref[...] = _softplus(z)
    return pl.pallas_call(
        f, out_shape=jax.ShapeDtypeStruct((N, D), jnp.float32),
    )(parts, x, g2l.reshape(1, D), be2l.reshape(1, D))


def _readout(x, W_after, b_after, W_h, b_h, W_out, b_out):
    def f(x_ref, wa_ref, ba_ref, wh_ref, bh_ref, wo_ref, bo_ref, o_ref):
        m = jnp.mean(x_ref[...], axis=0, keepdims=True)
        h = jnp.dot(m, wa_ref[...],
                    preferred_element_type=jnp.float32) + ba_ref[...]
        h = _softplus(h)
        for j in range(2):
            h = jnp.dot(h, wh_ref[j],
                        preferred_element_type=jnp.float32) + bh_ref[j:j + 1, :]
            h = _softplus(h)
        o_ref[...] = jnp.dot(h, wo_ref[...],
                             preferred_element_type=jnp.float32) + bo_ref[...]
    return pl.pallas_call(
        f, out_shape=jax.ShapeDtypeStruct((1, 1), jnp.float32),
    )(x, W_after, b_after.reshape(1, D), W_h, b_h, W_out, b_out.reshape(1, 1))


def _sc_gather2(A, B, dst, src):
    mesh = plsc.VectorSubcoreMesh(core_axis_name="c", subcore_axis_name="s")

    @functools.partial(
        pl.kernel, mesh=mesh,
        out_type=[jax.ShapeDtypeStruct((E, D), jnp.int32)] * 2,
        scratch_types=[pltpu.VMEM((PER_TILE,), jnp.int32),
                       pltpu.VMEM((PER_TILE,), jnp.int32),
                       pltpu.VMEM((CB, D), jnp.int32),
                       pltpu.VMEM((CB, D), jnp.int32),
                       pltpu.VMEM((CB, D), jnp.int32),
                       pltpu.VMEM((CB, D), jnp.int32),
                       pltpu.SemaphoreType.DMA,
                       pltpu.SemaphoreType.DMA,
                       pltpu.SemaphoreType.DMA,
                       pltpu.SemaphoreType.DMA],
    )
    def k(a_hbm, b_hbm, dst_hbm, src_hbm, o1_hbm, o2_hbm,
          di_v, si_v, ra0, rb0, ra1, rb1, sa0, sb0, sa1, sb1):
        wid = lax.axis_index("s") * NC + lax.axis_index("c")
        base = wid * PER_TILE
        pltpu.sync_copy(dst_hbm.at[pl.ds(base, PER_TILE)], di_v)
        pltpu.sync_copy(src_hbm.at[pl.ds(base, PER_TILE)], si_v)

        def start(ci, ra, rb, sa, sb):
            loc = ci * CB
            ca = pltpu.async_copy(a_hbm.at[di_v.at[pl.ds(loc, CB)]], ra, sa)
            cb = pltpu.async_copy(b_hbm.at[si_v.at[pl.ds(loc, CB)]], rb, sb)
            return ca, cb

        def finish(ci, ra, rb, sa, sb):
            pltpu.make_async_copy(a_hbm.at[di_v.at[pl.ds(0, CB)]], ra, sa).wait()
            pltpu.make_async_copy(b_hbm.at[si_v.at[pl.ds(0, CB)]], rb, sb).wait()
            off = base + ci * CB
            pltpu.sync_copy(ra, o1_hbm.at[pl.ds(off, CB)])
            pltpu.sync_copy(rb, o2_hbm.at[pl.ds(off, CB)])

        start(0, ra0, rb0, sa0, sb0)

        @pl.loop(0, (N_CHUNKS - 1) // 2)
        def _(kk):
            c = 2 * kk
            start(c + 1, ra1, rb1, sa1, sb1)
            finish(c, ra0, rb0, sa0, sb0)
            start(c + 2, ra0, rb0, sa0, sb0)
            finish(c + 1, ra1, rb1, sa1, sb1)

        finish(N_CHUNKS - 1, ra0, rb0, sa0, sb0)

    return k(A, B, dst, src)


MB = 80
N_MB = PER_TILE // MB


def _sc_scatter_add(msg, dst3):
    mesh = plsc.VectorSubcoreMesh(core_axis_name="c", subcore_axis_name="s")

    @functools.partial(
        pl.kernel, mesh=mesh,
        out_type=jax.ShapeDtypeStruct((NC, NP, D), jnp.float32),
        scratch_types=[pltpu.VMEM((N_CHUNKS, CB), jnp.int32),
                       pltpu.VMEM((MB, D), jnp.float32),
                       pltpu.VMEM((MB, D), jnp.float32),
                       pltpu.VMEM((16, D), jnp.float32),
                       pltpu.VMEM_SHARED((NP, D), jnp.float32),
                       pltpu.SemaphoreType.DMA,
                       pltpu.SemaphoreType.DMA],
    )
    def k(msg_hbm, dst_hbm, out_hbm, di_v, m0, m1, zv, agg_sh, sm0, sm1):
        c = lax.axis_index("c")
        s = lax.axis_index("s")
        wid = s * NC + c
        base = wid * PER_TILE
        pltpu.sync_copy(dst_hbm.at[wid], di_v)

        @pl.loop(0, 16)
        def _(i):
            @pl.loop(0, D // 16)
            def _(j):
                zv[i, pl.ds(j * 16, 16)] = jnp.zeros((16,), jnp.float32)

        @pl.loop(0, ROWS_PER_TILE // 16)
        def _(kk):
            pltpu.sync_copy(zv, agg_sh.at[pl.ds(s * ROWS_PER_TILE + kk * 16, 16)])

        plsc.subcore_barrier()

        def load(mb, buf, sem):
            pltpu.async_copy(msg_hbm.at[pl.ds(base + mb * MB, MB)], buf, sem)

        def drain(buf, sem):
            pltpu.make_async_copy(msg_hbm.at[pl.ds(base, MB)], buf, sem).wait()

        def scat(mb, buf):
            for j in range(MB // CB):
                pltpu.sync_copy(buf.at[pl.ds(j * CB, CB)],
                                agg_sh.at[di_v.at[mb * (MB // CB) + j]],
                                add=True)

        load(0, m0, sm0)

        @pl.loop(0, (N_MB - 1) // 2)
        def _(kk):
            mb = 2 * kk
            load(mb + 1, m1, sm1)
            drain(m0, sm0)
            scat(mb, m0)
            load(mb + 2, m0, sm0)
            drain(m1, sm1)
            scat(mb + 1, m1)

        drain(m0, sm0)
        scat(N_MB - 1, m0)

        plsc.subcore_barrier()

        pltpu.sync_copy(agg_sh.at[pl.ds(s * ROWS_PER_TILE, ROWS_PER_TILE)],
                        out_hbm.at[c, pl.ds(s * ROWS_PER_TILE, ROWS_PER_TILE)])

    return k(msg, dst3)


def kernel(n_inputs, e_inputs, edge_index, W_before, b_before, W_conv, b_conv,
           g1, be1, g2, be2, W_after, b_after, W_h, b_h, W_out, b_out):
    src = edge_index[0]
    dst = edge_index[1]
    dst3 = dst.reshape(NW, N_CHUNKS, CB)
    x = _linear_before(n_inputs, W_before, b_before)
    for l in range(3):
        Wd = W_conv[l, :D]
        Ws = W_conv[l, D:2 * D]
        We = W_conv[l, 2 * D:]
        A, B = _node_tables(x, Wd, Ws, b_conv[l])
        G1, G2 = _sc_gather2(A, B, dst, src)
        sums, Mp = _edge_stats(G1, G2, e_inputs, We)
        msg = _edge_messages(Mp, sums, g1[l], be1[l])
        parts = _sc_scatter_add(msg, dst3)
        x = _node_update(parts, x, g2[l], be2[l])
    return _readout(x, W_after, b_after, W_h, b_h, W_out, b_out)


# speedup vs baseline: 3.6574x; 1.0157x over previous
import functools

import jax
import jax.numpy as jnp
from jax import lax
from jax.experimental import pallas as pl
from jax.experimental.pallas import tpu as pltpu
from jax.experimental.pallas import tpu_sc as plsc

N = 10000
E = 320000
E1 = 192000
E2 = E - E1
D = 128
C2 = 256
ED = 16
EPS = 1e-5
NC, NS = 2, 16
NW = NC * NS
CB = 80
NP = 10240
ROWS_PER_TILE = NP // NS
EB = 4000


def _softplus(z):
    return jnp.maximum(z, 0.0) + jnp.log1p(jnp.exp(-jnp.abs(z)))


def _pack2(f, c):
    fb = lax.bitcast_convert_type(f, jnp.int32)
    cb = lax.bitcast_convert_type(c, jnp.int32)
    return jnp.bitwise_or(jnp.bitwise_and(cb, jnp.int32(-65536)),
                          lax.shift_right_logical(fb, 16))


def _unpack16(w):
    f = lax.bitcast_convert_type(lax.shift_left(w, 16), jnp.float32)
    c = lax.bitcast_convert_type(jnp.bitwise_and(w, jnp.int32(-65536)),
                                 jnp.float32)
    return f, c


def _linear_before(n_inputs, W, b):
    def f(x_ref, w_ref, b_ref, o_ref):
        o_ref[...] = jnp.dot(x_ref[...], w_ref[...],
                             preferred_element_type=jnp.float32) + b_ref[...]
    return pl.pallas_call(
        f, out_shape=jax.ShapeDtypeStruct((N, D), jnp.float32),
    )(n_inputs, W, b.reshape(1, D))


def _node_tables(x, Wd, Ws, b):
    def f(x_ref, wd_ref, ws_ref, b_ref, a_ref, bb_ref):
        xv = x_ref[...]
        a = jnp.dot(xv, wd_ref[...],
                    preferred_element_type=jnp.float32) + b_ref[...]
        bb = jnp.dot(xv, ws_ref[...], preferred_element_type=jnp.float32)
        a_ref[...] = _pack2(a[:, :D], a[:, D:])
        bb_ref[...] = _pack2(bb[:, :D], bb[:, D:])
    return pl.pallas_call(
        f, out_shape=[jax.ShapeDtypeStruct((N, D), jnp.int32)] * 2,
    )(x, Wd, Ws, b.reshape(1, C2))


def _edge_m(g1_ref, g2_ref, e_ref, we_ref):
    fA, cA = _unpack16(g1_ref[...])
    fB, cB = _unpack16(g2_ref[...])
    cc = jnp.dot(e_ref[...], we_ref[...], preferred_element_type=jnp.float32)
    m_f = fA + fB + cc[:, :D]
    m_c = cA + cB + cc[:, D:]
    return m_f, m_c


def _edge_stats(G1, G2, e_inputs, We):
    e_len = G1.shape[0]

    def f(g1_ref, g2_ref, e_ref, we_ref, acc_ref, mp_ref):
        i = pl.program_id(0)
        m_f, m_c = _edge_m(g1_ref, g2_ref, e_ref, we_ref)
        mp_ref[...] = _pack2(m_f, m_c)
        upd = jnp.concatenate(
            [jnp.sum(m_f, axis=0, keepdims=True),
             jnp.sum(m_c, axis=0, keepdims=True),
             jnp.sum(m_f * m_f, axis=0, keepdims=True),
             jnp.sum(m_c * m_c, axis=0, keepdims=True),
             jnp.zeros((4, D), jnp.float32)], axis=0)

        @pl.when(i == 0)
        def _():
            acc_ref[...] = jnp.zeros_like(acc_ref)

        acc_ref[...] += upd

    return pl.pallas_call(
        f,
        grid=(e_len // EB,),
        in_specs=[pl.BlockSpec((EB, D), lambda i: (i, 0)),
                  pl.BlockSpec((EB, D), lambda i: (i, 0)),
                  pl.BlockSpec((EB, ED), lambda i: (i, 0)),
                  pl.BlockSpec((ED, C2), lambda i: (0, 0))],
        out_specs=[pl.BlockSpec((8, D), lambda i: (0, 0)),
                   pl.BlockSpec((EB, D), lambda i: (i, 0))],
        out_shape=[jax.ShapeDtypeStruct((8, D), jnp.float32),
                   jax.ShapeDtypeStruct((e_len, D), jnp.int32)],
    )(G1, G2, e_inputs, We)


def _edge_messages(Mp, sums_a, sums_b, g1l, be1l):
    e_len = Mp.shape[0]

    def f(mp_ref, sa_ref, sb_ref, ga_ref, be_ref, o_ref):
        s = sa_ref[...] + sb_ref[...]
        inv_e = 1.0 / E
        mu_f = s[0:1, :] * inv_e
        mu_c = s[1:2, :] * inv_e
        var_f = s[2:3, :] * inv_e - mu_f * mu_f
        var_c = s[3:4, :] * inv_e - mu_c * mu_c
        sc_f = ga_ref[0:1, :] * lax.rsqrt(var_f + EPS)
        sc_c = ga_ref[1:2, :] * lax.rsqrt(var_c + EPS)
        sh_f = be_ref[0:1, :] - mu_f * sc_f
        sh_c = be_ref[1:2, :] - mu_c * sc_c
        m_f, m_c = _unpack16(mp_ref[...])
        filt = m_f * sc_f + sh_f
        core = m_c * sc_c + sh_c
        sg = 1.0 / (1.0 + jnp.exp(-filt))
        o_ref[...] = _softplus(core) * sg

    return pl.pallas_call(
        f,
        grid=(e_len // EB,),
        in_specs=[pl.BlockSpec((EB, D), lambda i: (i, 0)),
                  pl.BlockSpec((8, D), lambda i: (0, 0)),
                  pl.BlockSpec((8, D), lambda i: (0, 0)),
                  pl.BlockSpec((2, D), lambda i: (0, 0)),
                  pl.BlockSpec((2, D), lambda i: (0, 0))],
        out_specs=pl.BlockSpec((EB, D), lambda i: (i, 0)),
        out_shape=jax.ShapeDtypeStruct((e_len, D), jnp.float32),
    )(Mp, sums_a, sums_b, g1l.reshape(2, D), be1l.reshape(2, D))


def _node_update(parts, x, g2l, be2l):
    def f(p_ref, x_ref, g_ref, b_ref, o_ref):
        agg = p_ref[0, :N, :] + p_ref[1, :N, :]
        mu = jnp.mean(agg, axis=0, keepdims=True)
        var = jnp.mean(agg * agg, axis=0, keepdims=True) - mu * mu
        z = ((agg - mu) * (g_ref[...] * lax.rsqrt(var + EPS))
             + b_ref[...] + x_ref[...])
        o_ref[...] = _softplus(z)
    return pl.pallas_call(
        f, out_shape=jax.ShapeDtypeStruct((N, D), jnp.float32),
    )(parts, x, g2l.reshape(1, D), be2l.reshape(1, D))


def _readout(x, W_after, b_after, W_h, b_h, W_out, b_out):
    def f(x_ref, wa_ref, ba_ref, wh_ref, bh_ref, wo_ref, bo_ref, o_ref):
        m = jnp.mean(x_ref[...], axis=0, keepdims=True)
        h = jnp.dot(m, wa_ref[...],
                    preferred_element_type=jnp.float32) + ba_ref[...]
        h = _softplus(h)
        for j in range(2):
            h = jnp.dot(h, wh_ref[j],
                        preferred_element_type=jnp.float32) + bh_ref[j:j + 1, :]
            h = _softplus(h)
        o_ref[...] = jnp.dot(h, wo_ref[...],
                             preferred_element_type=jnp.float32) + bo_ref[...]
    return pl.pallas_call(
        f, out_shape=jax.ShapeDtypeStruct((1, 1), jnp.float32),
    )(x, W_after, b_after.reshape(1, D), W_h, b_h, W_out, b_out.reshape(1, 1))


def _sc_gather2(A, B, dst, src, e_off, e_len):
    per_tile = e_len // NW
    n_chunks = per_tile // CB
    mesh = plsc.VectorSubcoreMesh(core_axis_name="c", subcore_axis_name="s")

    @functools.partial(
        pl.kernel, mesh=mesh,
        out_type=[jax.ShapeDtypeStruct((e_len, D), jnp.int32)] * 2,
        scratch_types=[pltpu.VMEM((per_tile,), jnp.int32),
                       pltpu.VMEM((per_tile,), jnp.int32),
                       pltpu.VMEM((CB, D), jnp.int32),
                       pltpu.VMEM((CB, D), jnp.int32),
                       pltpu.VMEM((CB, D), jnp.int32),
                       pltpu.VMEM((CB, D), jnp.int32),
                       pltpu.SemaphoreType.DMA,
                       pltpu.SemaphoreType.DMA,
                       pltpu.SemaphoreType.DMA,
                       pltpu.SemaphoreType.DMA],
    )
    def k(a_hbm, b_hbm, dst_hbm, src_hbm, o1_hbm, o2_hbm,
          di_v, si_v, ra0, rb0, ra1, rb1, sa0, sb0, sa1, sb1):
        wid = lax.axis_index("s") * NC + lax.axis_index("c")
        base = wid * per_tile
        pltpu.sync_copy(dst_hbm.at[pl.ds(e_off + base, per_tile)], di_v)
        pltpu.sync_copy(src_hbm.at[pl.ds(e_off + base, per_tile)], si_v)

        def start(ci, ra, rb, sa, sb):
            loc = ci * CB
            pltpu.async_copy(a_hbm.at[di_v.at[pl.ds(loc, CB)]], ra, sa)
            pltpu.async_copy(b_hbm.at[si_v.at[pl.ds(loc, CB)]], rb, sb)

        def finish(ci, ra, rb, sa, sb):
            pltpu.make_async_copy(a_hbm.at[di_v.at[pl.ds(0, CB)]], ra, sa).wait()
            pltpu.make_async_copy(b_hbm.at[si_v.at[pl.ds(0, CB)]], rb, sb).wait()
            off = base + ci * CB
            pltpu.sync_copy(ra, o1_hbm.at[pl.ds(off, CB)])
            pltpu.sync_copy(rb, o2_hbm.at[pl.ds(off, CB)])

        start(0, ra0, rb0, sa0, sb0)

        n_pairs = (n_chunks - 1) // 2 if n_chunks % 2 else (n_chunks - 2) // 2

        @pl.loop(0, n_pairs)
        def _(kk):
            c = 2 * kk
            start(c + 1, ra1, rb1, sa1, sb1)
            finish(c, ra0, rb0, sa0, sb0)
            start(c + 2, ra0, rb0, sa0, sb0)
            finish(c + 1, ra1, rb1, sa1, sb1)

        if n_chunks % 2:
            finish(n_chunks - 1, ra0, rb0, sa0, sb0)
        else:
            start(n_chunks - 1, ra1, rb1, sa1, sb1)
            finish(n_chunks - 2, ra0, rb0, sa0, sb0)
            finish(n_chunks - 1, ra1, rb1, sa1, sb1)

    return k(A, B, dst, src)


def _sc_scatter_add(msg, dst3, prev):
    per_tile = msg.shape[0] // NW
    n_chunks = per_tile // CB
    mesh = plsc.VectorSubcoreMesh(core_axis_name="c", subcore_axis_name="s")

    scratch = [pltpu.VMEM((n_chunks, CB), jnp.int32),
               pltpu.VMEM((CB, D), jnp.float32),
               pltpu.VMEM((CB, D), jnp.float32)]
    if prev is None:
        scratch.append(pltpu.VMEM((16, D), jnp.float32))
    scratch += [pltpu.VMEM_SHARED((NP, D), jnp.float32),
                pltpu.SemaphoreType.DMA,
                pltpu.SemaphoreType.DMA]

    @functools.partial(
        pl.kernel, mesh=mesh,
        out_type=jax.ShapeDtypeStruct((NC, NP, D), jnp.float32),
        scratch_types=scratch,
    )
    def k(msg_hbm, dst_hbm, *rest):
        if prev is None:
            out_hbm, di_v, m0, m1, zv, agg_sh, sm0, sm1 = rest
        else:
            prev_hbm, out_hbm, di_v, m0, m1, agg_sh, sm0, sm1 = rest
        c = lax.axis_index("c")
        s = lax.axis_index("s")
        wid = s * NC + c
        base = wid * per_tile
        pltpu.sync_copy(dst_hbm.at[wid], di_v)

        rows = pl.ds(s * ROWS_PER_TILE, ROWS_PER_TILE)
        if prev is None:
            @pl.loop(0, 16)
            def _(i):
                @pl.loop(0, D // 16)
                def _(j):
                    zv[i, pl.ds(j * 16, 16)] = jnp.zeros((16,), jnp.float32)

            @pl.loop(0, ROWS_PER_TILE // 16)
            def _(kk):
                pltpu.sync_copy(
                    zv, agg_sh.at[pl.ds(s * ROWS_PER_TILE + kk * 16, 16)])
        else:
            pltpu.sync_copy(prev_hbm.at[c, rows], agg_sh.at[rows])

        plsc.subcore_barrier()

        def load(mb, buf, sem):
            pltpu.async_copy(msg_hbm.at[pl.ds(base + mb * CB, CB)], buf, sem)

        def drain(buf, sem):
            pltpu.make_async_copy(msg_hbm.at[pl.ds(base, CB)], buf, sem).wait()

        def scat(mb, buf):
            pltpu.sync_copy(buf, agg_sh.at[di_v.at[mb]], add=True)

        load(0, m0, sm0)

        n_pairs = (n_chunks - 1) // 2 if n_chunks % 2 else (n_chunks - 2) // 2

        @pl.loop(0, n_pairs)
        def _(kk):
            mb = 2 * kk
            load(mb + 1, m1, sm1)
            drain(m0, sm0)
            scat(mb, m0)
            load(mb + 2, m0, sm0)
            drain(m1, sm1)
            scat(mb + 1, m1)

        if n_chunks % 2:
            drain(m0, sm0)
            scat(n_chunks - 1, m0)
        else:
            load(n_chunks - 1, m1, sm1)
            drain(m0, sm0)
            scat(n_chunks - 2, m0)
            drain(m1, sm1)
            scat(n_chunks - 1, m1)

        plsc.subcore_barrier()

        pltpu.sync_copy(agg_sh.at[rows], out_hbm.at[c, rows])

    if prev is None:
        return k(msg, dst3)
    return k(msg, dst3, prev)


def kernel(n_inputs, e_inputs, edge_index, W_before, b_before, W_conv, b_conv,
           g1, be1, g2, be2, W_after, b_after, W_h, b_h, W_out, b_out):
    src = edge_index[0]
    dst = edge_index[1]
    dst3a = dst[:E1].reshape(NW, E1 // NW // CB, CB)
    dst3b = dst[E1:].reshape(NW, E2 // NW // CB, CB)
    e_a = e_inputs[:E1]
    e_b = e_inputs[E1:]
    x = _linear_before(n_inputs, W_before, b_before)
    for l in range(3):
        Wd = W_conv[l, :D]
        Ws = W_conv[l, D:2 * D]
        We = W_conv[l, 2 * D:]
        A, B = _node_tables(x, Wd, Ws, b_conv[l])
        G1a, G2a = _sc_gather2(A, B, dst, src, 0, E1)
        G1b, G2b = _sc_gather2(A, B, dst, src, E1, E2)
        sums_a, Mpa = _edge_stats(G1a, G2a, e_a, We)
        sums_b, Mpb = _edge_stats(G1b, G2b, e_b, We)
        msg_a = _edge_messages(Mpa, sums_a, sums_b, g1[l], be1[l])
        msg_b = _edge_messages(Mpb, sums_a, sums_b, g1[l], be1[l])
        p1 = _sc_scatter_add(msg_a, dst3a, None)
        parts = _sc_scatter_add(msg_b, dst3b, p1)
        x = _node_update(parts, x, g2[l], be2[l])
    return _readout(x, W_after, b_after, W_h, b_h, W_out, b_out)
